# baseline (speedup 1.0000x reference)
import functools

import jax
import jax.numpy as jnp
from jax import lax
from jax.experimental import pallas as pl
from jax.experimental.pallas import tpu as pltpu
from jax.experimental.pallas import tpu_sc as plsc

NTYPES = 8
NLOC = 10000
NNEI = 64
ND = 128
ED = 64
RCUT = 6.0
RCUT_SMTH = 5.0
EPS = 1e-6
CPAD = 16

E = NLOC * NNEI
BN = 80
BE = BN * NNEI
NB = NLOC // BN

RW = 80
NC, NS = 2, 16
NW = NC * NS
RPW = 256
EP = NW * RPW * RW


def _silu(x):
    return x / (1.0 + jnp.exp(-x))


def _dot(a, b):
    return jnp.dot(a, b, preferred_element_type=jnp.float32)


def _sc_gather(table, idx3):
    mesh = plsc.VectorSubcoreMesh(core_axis_name="c", subcore_axis_name="s")

    @functools.partial(
        pl.kernel,
        mesh=mesh,
        out_type=jax.ShapeDtypeStruct((EP, ND), jnp.float32),
        scratch_types=[
            pltpu.VMEM((RPW, RW), jnp.int32),
            pltpu.VMEM((RW, ND), jnp.float32),
            pltpu.SemaphoreType.DMA,
        ],
    )
    def gk(table_hbm, idx_hbm, out_hbm, idx_v, row_v, sem):
        wid = lax.axis_index("s") * NC + lax.axis_index("c")
        base = wid * RPW
        pltpu.sync_copy(idx_hbm.at[wid], idx_v)

        def body(j, carry):
            pltpu.async_copy(table_hbm.at[idx_v.at[j]], row_v, sem).wait()
            pltpu.sync_copy(row_v, out_hbm.at[pl.ds((base + j) * RW, RW)])
            return carry

        lax.fori_loop(0, RPW, body, 0)

    return gk(table, idx3)


def _stage0_body(at_ref, tt_ref, w2_ref, ne_ref, p_ref):
    a = at_ref[:]
    ne = jnp.zeros((BN, ND), jnp.float32)
    for t in range(NTYPES):
        sel = (a == t).astype(jnp.float32)
        ne = ne + sel * tt_ref[t:t + 1, :]
    ne_ref[:] = ne
    p_ref[:] = _dot(ne, w2_ref[:])


def _stage0(at2, type_table, w2):
    return pl.pallas_call(
        _stage0_body,
        grid=(NB,),
        in_specs=[
            pl.BlockSpec((BN, 1), lambda i: (i, 0)),
            pl.BlockSpec((NTYPES, ND), lambda i: (0, 0)),
            pl.BlockSpec((ND, ED), lambda i: (0, 0)),
        ],
        out_specs=[
            pl.BlockSpec((BN, ND), lambda i: (i, 0)),
            pl.BlockSpec((BN, ED), lambda i: (i, 0)),
        ],
        out_shape=[
            jax.ShapeDtypeStruct((NLOC, ND), jnp.float32),
            jax.ShapeDtypeStruct((NLOC, ED), jnp.float32),
        ],
    )(at2, type_table, w2)


def _geom_body(t_ref, c_ref, we_ref, be_ref, e_ref, sw_ref):
    R3 = t_ref[:, ED:ED + CPAD].reshape(BN, NNEI, CPAD)
    d2 = lax.dot_general(R3, c_ref[:], (((2,), (1,)), ((0,), (0,))),
                         preferred_element_type=jnp.float32)
    dist2 = jnp.sqrt(jnp.maximum(d2, 0.0))
    uu = jnp.clip((dist2 - RCUT_SMTH) / (RCUT - RCUT_SMTH), 0.0, 1.0)
    sw2 = uu * uu * uu * (-6.0 * uu * uu + 15.0 * uu - 10.0) + 1.0
    dist = dist2.reshape(BN, NNEI, 1)
    sw = sw2.reshape(BN, NNEI, 1)
    we3 = we_ref[:].reshape(1, 1, ED)
    be3 = be_ref[:].reshape(1, 1, ED)
    e0 = _silu(dist * we3 + be3) * sw
    e_ref[:] = e0.reshape(BE, ED)
    sw_ref[:] = sw2.reshape(BE, 1)


def _geom(t0g, cgeo, we0, be0):
    return pl.pallas_call(
        _geom_body,
        grid=(NB,),
        in_specs=[
            pl.BlockSpec((BE, ND), lambda i: (i, 0)),
            pl.BlockSpec((BN, CPAD), lambda i: (i, 0)),
            pl.BlockSpec((1, ED), lambda i: (0, 0)),
            pl.BlockSpec((1, ED), lambda i: (0, 0)),
        ],
        out_specs=[
            pl.BlockSpec((BE, ED), lambda i: (i, 0)),
            pl.BlockSpec((BE, 1), lambda i: (i, 0)),
        ],
        out_shape=[
            jax.ShapeDtypeStruct((E, ED), jnp.float32),
            jax.ShapeDtypeStruct((E, 1), jnp.float32),
        ],
    )(t0g, cgeo, we0, be0)


def _edge_body(last, g_direct, e_ref, nb_ref, n_ref, sw_ref, w1_ref, w2_ref,
               w3_ref, wn1_ref, wn2_ref, *out_refs):
    e = e_ref[:]
    sw = sw_ref[:]
    if g_direct:
        pre = _dot(e, w3_ref[:]) + nb_ref[:, :ED]
    else:
        pre = _dot(e, w3_ref[:]) + _dot(nb_ref[:], w2_ref[:])
    n = n_ref[:]
    a = _dot(n, w1_ref[:])
    sw3 = sw.reshape(BN, NNEI, 1)
    pre3 = pre.reshape(BN, NNEI, ED) + a[:, None, :]
    e3 = e.reshape(BN, NNEI, ED) + _silu(pre3) * sw3
    if last:
        (no_ref,) = out_refs
    else:
        eo_ref, no_ref = out_refs
        eo_ref[:] = e3.reshape(BE, ED)
    msg = jnp.sum(e3 * sw3, axis=1) * (1.0 / NNEI)
    h = _dot(n, wn1_ref[:]) + _dot(msg, wn2_ref[:])
    no_ref[:] = n + _silu(h)


def _edge(e, nbg, node, sw, w1, w2, w3, wn1, wn2, last, g_direct=False):
    out_specs = [pl.BlockSpec((BN, ND), lambda i: (i, 0))]
    out_shape = [jax.ShapeDtypeStruct((NLOC, ND), jnp.float32)]
    if not last:
        out_specs.insert(0, pl.BlockSpec((BE, ED), lambda i: (i, 0)))
        out_shape.insert(0, jax.ShapeDtypeStruct((E, ED), jnp.float32))
    return pl.pallas_call(
        functools.partial(_edge_body, last, g_direct),
        grid=(NB,),
        in_specs=[
            pl.BlockSpec((BE, ED), lambda i: (i, 0)),
            pl.BlockSpec((BE, ND), lambda i: (i, 0)),
            pl.BlockSpec((BN, ND), lambda i: (i, 0)),
            pl.BlockSpec((BE, 1), lambda i: (i, 0)),
            pl.BlockSpec((ND, ED), lambda i: (0, 0)),
            pl.BlockSpec((ND, ED), lambda i: (0, 0)),
            pl.BlockSpec((ED, ED), lambda i: (0, 0)),
            pl.BlockSpec((ND, ND), lambda i: (0, 0)),
            pl.BlockSpec((ED, ND), lambda i: (0, 0)),
        ],
        out_specs=out_specs,
        out_shape=out_shape,
    )(e, nbg, node, sw, w1, w2, w3, wn1, wn2)


def kernel(extended_coord, extended_atype, nlist, mapping, type_table,
           W_e0, b_e0, W_node, W_edge):
    coords = extended_coord[0].astype(jnp.float32)
    ones = jnp.ones((NLOC, 1), jnp.float32)
    zeros11 = jnp.zeros((NLOC, CPAD - 5), jnp.float32)
    cn2 = jnp.sum(coords * coords, axis=1, keepdims=True)
    csum = jnp.sum(coords, axis=1, keepdims=True)
    rgeo = jnp.concatenate([coords, cn2, ones, zeros11], axis=1)
    cgeo = jnp.concatenate(
        [-2.0 * coords + 2.0 * EPS, ones,
         cn2 - 2.0 * EPS * csum + 3.0 * EPS * EPS, zeros11], axis=1)
    at2 = extended_atype[0].astype(jnp.int32).reshape(NLOC, 1)
    idx_flat = nlist[0].astype(jnp.int32).reshape(-1)
    idx3 = jnp.concatenate(
        [idx_flat, jnp.zeros((EP - E,), jnp.int32)]).reshape(NW, RPW, RW)
    W1 = W_edge[:, :ND, :]
    W2 = W_edge[:, ND:2 * ND, :]
    W3 = W_edge[:, 2 * ND:, :]
    Wn1 = W_node[:, :ND, :]
    Wn2 = W_node[:, ND:, :]
    we0 = W_e0.reshape(1, ED)
    be0 = b_e0.reshape(1, ED)

    node0, p0 = _stage0(at2, type_table, W2[0])
    t0 = jnp.concatenate(
        [p0, rgeo, jnp.zeros((NLOC, ND - ED - CPAD), jnp.float32)], axis=1)
    t0g = _sc_gather(t0, idx3)
    e0, sw = _geom(t0g, cgeo, we0, be0)
    e1, node1 = _edge(e0, t0g, node0, sw, W1[0], W2[0], W3[0],
                      Wn1[0], Wn2[0], last=False, g_direct=True)
    nb1 = _sc_gather(node1, idx3)
    e2, node2 = _edge(e1, nb1, node1, sw, W1[1], W2[1], W3[1],
                      Wn1[1], Wn2[1], last=False)
    nb2 = _sc_gather(node2, idx3)
    (node3,) = _edge(e2, nb2, node2, sw, W1[2], W2[2], W3[2],
                     Wn1[2], Wn2[2], last=True)
    return node3[None]


# speedup vs baseline: 2.2253x; 2.2253x over previous
import functools

import jax
import jax.numpy as jnp
from jax import lax
from jax.experimental import pallas as pl
from jax.experimental.pallas import tpu as pltpu
from jax.experimental.pallas import tpu_sc as plsc

NTYPES = 8
NLOC = 10000
NNEI = 64
ND = 128
ED = 64
RCUT = 6.0
RCUT_SMTH = 5.0
EPS = 1e-6
CPAD = 16

E = NLOC * NNEI
BN = 80
BE = BN * NNEI
NB = NLOC // BN

RW = 80
NC, NS = 2, 16
NW = NC * NS
RPW = E // (NW * RW)
EP = E


def _silu(x):
    return x / (1.0 + jnp.exp(-x))


def _dot(a, b):
    return jnp.dot(a, b, preferred_element_type=jnp.float32)


def _sc_gather(table, idx3):
    mesh = plsc.VectorSubcoreMesh(core_axis_name="c", subcore_axis_name="s")

    @functools.partial(
        pl.kernel,
        mesh=mesh,
        out_type=jax.ShapeDtypeStruct((EP, ND), jnp.float32),
        scratch_types=[
            pltpu.VMEM((RPW, RW), jnp.int32),
            pltpu.VMEM((RW, ND), jnp.float32),
            pltpu.SemaphoreType.DMA,
        ],
    )
    def gk(table_hbm, idx_hbm, out_hbm, idx_v, row_v, sem):
        wid = lax.axis_index("s") * NC + lax.axis_index("c")
        base = wid * RPW
        pltpu.sync_copy(idx_hbm.at[wid], idx_v)

        def body(j, carry):
            pltpu.async_copy(table_hbm.at[idx_v.at[j]], row_v, sem).wait()
            pltpu.sync_copy(row_v, out_hbm.at[pl.ds((base + j) * RW, RW)])
            return carry

        lax.fori_loop(0, RPW, body, 0)

    return gk(table, idx3)


def _stage0_body(at_ref, tt_ref, w2_ref, ne_ref, p_ref):
    a = at_ref[:]
    ne = jnp.zeros((BN, ND), jnp.float32)
    for t in range(NTYPES):
        sel = (a == t).astype(jnp.float32)
        ne = ne + sel * tt_ref[t:t + 1, :]
    ne_ref[:] = ne
    p_ref[:] = _dot(ne, w2_ref[:])


def _stage0(at2, type_table, w2):
    return pl.pallas_call(
        _stage0_body,
        grid=(NB,),
        in_specs=[
            pl.BlockSpec((BN, 1), lambda i: (i, 0)),
            pl.BlockSpec((NTYPES, ND), lambda i: (0, 0)),
            pl.BlockSpec((ND, ED), lambda i: (0, 0)),
        ],
        out_specs=[
            pl.BlockSpec((BN, ND), lambda i: (i, 0)),
            pl.BlockSpec((BN, ED), lambda i: (i, 0)),
        ],
        out_shape=[
            jax.ShapeDtypeStruct((NLOC, ND), jnp.float32),
            jax.ShapeDtypeStruct((NLOC, ED), jnp.float32),
        ],
    )(at2, type_table, w2)


def _geom_body(t_ref, c_ref, we_ref, be_ref, e_ref, sw_ref):
    R3 = t_ref[:, ED:ED + CPAD].reshape(BN, NNEI, CPAD)
    C3 = c_ref[:].reshape(BN, CPAD, 1)
    d2 = lax.dot_general(R3, C3, (((2,), (1,)), ((0,), (0,))),
                         preferred_element_type=jnp.float32)
    dist = jnp.sqrt(jnp.maximum(d2, 0.0))
    uu = jnp.clip((dist - RCUT_SMTH) / (RCUT - RCUT_SMTH), 0.0, 1.0)
    sw = uu * uu * uu * (-6.0 * uu * uu + 15.0 * uu - 10.0) + 1.0
    we3 = we_ref[:].reshape(1, 1, ED)
    be3 = be_ref[:].reshape(1, 1, ED)
    e0 = _silu(dist * we3 + be3) * sw
    e_ref[:] = e0.reshape(BE, ED)
    sw_ref[:] = sw.reshape(BE, 1)


def _geom(t0g, cgeo, we0, be0):
    return pl.pallas_call(
        _geom_body,
        grid=(NB,),
        in_specs=[
            pl.BlockSpec((BE, ND), lambda i: (i, 0)),
            pl.BlockSpec((BN, CPAD), lambda i: (i, 0)),
            pl.BlockSpec((1, ED), lambda i: (0, 0)),
            pl.BlockSpec((1, ED), lambda i: (0, 0)),
        ],
        out_specs=[
            pl.BlockSpec((BE, ED), lambda i: (i, 0)),
            pl.BlockSpec((BE, 1), lambda i: (i, 0)),
        ],
        out_shape=[
            jax.ShapeDtypeStruct((E, ED), jnp.float32),
            jax.ShapeDtypeStruct((E, 1), jnp.float32),
        ],
    )(t0g, cgeo, we0, be0)


def _edge_body(last, g_direct, e_ref, nb_ref, n_ref, sw_ref, w1_ref, w2_ref,
               w3_ref, wn1_ref, wn2_ref, *out_refs):
    e = e_ref[:]
    sw = sw_ref[:]
    if g_direct:
        pre = _dot(e, w3_ref[:]) + nb_ref[:, :ED]
    else:
        pre = _dot(e, w3_ref[:]) + _dot(nb_ref[:], w2_ref[:])
    n = n_ref[:]
    a = _dot(n, w1_ref[:])
    sw3 = sw.reshape(BN, NNEI, 1)
    pre3 = pre.reshape(BN, NNEI, ED) + a[:, None, :]
    e3 = e.reshape(BN, NNEI, ED) + _silu(pre3) * sw3
    if last:
        (no_ref,) = out_refs
    else:
        eo_ref, no_ref = out_refs
        eo_ref[:] = e3.reshape(BE, ED)
    msg = jnp.sum(e3 * sw3, axis=1) * (1.0 / NNEI)
    h = _dot(n, wn1_ref[:]) + _dot(msg, wn2_ref[:])
    no_ref[:] = n + _silu(h)


def _edge(e, nbg, node, sw, w1, w2, w3, wn1, wn2, last, g_direct=False):
    out_specs = [pl.BlockSpec((BN, ND), lambda i: (i, 0))]
    out_shape = [jax.ShapeDtypeStruct((NLOC, ND), jnp.float32)]
    if not last:
        out_specs.insert(0, pl.BlockSpec((BE, ED), lambda i: (i, 0)))
        out_shape.insert(0, jax.ShapeDtypeStruct((E, ED), jnp.float32))
    return pl.pallas_call(
        functools.partial(_edge_body, last, g_direct),
        grid=(NB,),
        in_specs=[
            pl.BlockSpec((BE, ED), lambda i: (i, 0)),
            pl.BlockSpec((BE, ND), lambda i: (i, 0)),
            pl.BlockSpec((BN, ND), lambda i: (i, 0)),
            pl.BlockSpec((BE, 1), lambda i: (i, 0)),
            pl.BlockSpec((ND, ED), lambda i: (0, 0)),
            pl.BlockSpec((ND, ED), lambda i: (0, 0)),
            pl.BlockSpec((ED, ED), lambda i: (0, 0)),
            pl.BlockSpec((ND, ND), lambda i: (0, 0)),
            pl.BlockSpec((ED, ND), lambda i: (0, 0)),
        ],
        out_specs=out_specs,
        out_shape=out_shape,
    )(e, nbg, node, sw, w1, w2, w3, wn1, wn2)


def kernel(extended_coord, extended_atype, nlist, mapping, type_table,
           W_e0, b_e0, W_node, W_edge):
    coords = extended_coord[0].astype(jnp.float32)
    ones = jnp.ones((NLOC, 1), jnp.float32)
    zeros11 = jnp.zeros((NLOC, CPAD - 5), jnp.float32)
    cn2 = jnp.sum(coords * coords, axis=1, keepdims=True)
    csum = jnp.sum(coords, axis=1, keepdims=True)
    rgeo = jnp.concatenate([coords, cn2, ones, zeros11], axis=1)
    cgeo = jnp.concatenate(
        [-2.0 * coords + 2.0 * EPS, ones,
         cn2 - 2.0 * EPS * csum + 3.0 * EPS * EPS, zeros11], axis=1)
    at2 = extended_atype[0].astype(jnp.int32).reshape(NLOC, 1)
    idx3 = nlist[0].astype(jnp.int32).reshape(NW, RPW, RW)
    W1 = W_edge[:, :ND, :]
    W2 = W_edge[:, ND:2 * ND, :]
    W3 = W_edge[:, 2 * ND:, :]
    Wn1 = W_node[:, :ND, :]
    Wn2 = W_node[:, ND:, :]
    we0 = W_e0.reshape(1, ED)
    be0 = b_e0.reshape(1, ED)

    node0, p0 = _stage0(at2, type_table, W2[0])
    t0 = jnp.concatenate(
        [p0, rgeo, jnp.zeros((NLOC, ND - ED - CPAD), jnp.float32)], axis=1)
    t0g = _sc_gather(t0, idx3)
    e0, sw = _geom(t0g, cgeo, we0, be0)
    e1, node1 = _edge(e0, t0g, node0, sw, W1[0], W2[0], W3[0],
                      Wn1[0], Wn2[0], last=False, g_direct=True)
    nb1 = _sc_gather(node1, idx3)
    e2, node2 = _edge(e1, nb1, node1, sw, W1[1], W2[1], W3[1],
                      Wn1[1], Wn2[1], last=False)
    nb2 = _sc_gather(node2, idx3)
    (node3,) = _edge(e2, nb2, node2, sw, W1[2], W2[2], W3[2],
                     Wn1[2], Wn2[2], last=True)
    return node3[None]


# speedup vs baseline: 2.6470x; 1.1895x over previous
import functools

import jax
import jax.numpy as jnp
from jax import lax
from jax.experimental import pallas as pl
from jax.experimental.pallas import tpu as pltpu
from jax.experimental.pallas import tpu_sc as plsc

NTYPES = 8
NLOC = 10000
NNEI = 64
ND = 128
ED = 64
RCUT = 6.0
RCUT_SMTH = 5.0
EPS = 1e-6
CPAD = 16

E = NLOC * NNEI
BN = 80
BE = BN * NNEI
NB = NLOC // BN

RW = 80
NC, NS = 2, 16
NW = NC * NS
RPW = E // (NW * RW)
EP = E


def _silu(x):
    return x / (1.0 + jnp.exp(-x))


def _dot(a, b):
    return jnp.dot(a, b, preferred_element_type=jnp.float32)


def _sc_gather(table, idx3):
    mesh = plsc.VectorSubcoreMesh(core_axis_name="c", subcore_axis_name="s")

    @functools.partial(
        pl.kernel,
        mesh=mesh,
        out_type=jax.ShapeDtypeStruct((EP, ND), jnp.float32),
        scratch_types=[
            pltpu.VMEM((RPW, RW), jnp.int32),
            pltpu.VMEM((2, RW, ND), jnp.float32),
            pltpu.SemaphoreType.DMA,
            pltpu.SemaphoreType.DMA,
        ],
    )
    def gk(table_hbm, idx_hbm, out_hbm, idx_v, row_v, sem0, sem1):
        wid = lax.axis_index("s") * NC + lax.axis_index("c")
        base = wid * RPW
        pltpu.sync_copy(idx_hbm.at[wid], idx_v)

        def gath(j, b, sem):
            pltpu.async_copy(table_hbm.at[idx_v.at[j]], row_v.at[b], sem)

        def gwait(b, sem):
            pltpu.make_async_copy(
                table_hbm.at[idx_v.at[0]], row_v.at[b], sem).wait()

        def scat(j, b):
            pltpu.sync_copy(
                row_v.at[b], out_hbm.at[pl.ds((base + j) * RW, RW)])

        gath(0, 0, sem0)

        def body(t, carry):
            j = 2 * t
            gath(j + 1, 1, sem1)
            gwait(0, sem0)
            scat(j, 0)
            gath(j + 2, 0, sem0)
            gwait(1, sem1)
            scat(j + 1, 1)
            return carry

        lax.fori_loop(0, RPW // 2 - 1, body, 0)
        gath(RPW - 1, 1, sem1)
        gwait(0, sem0)
        scat(RPW - 2, 0)
        gwait(1, sem1)
        scat(RPW - 1, 1)

    return gk(table, idx3)


def _stage0_body(at_ref, tt_ref, w2_ref, ne_ref, p_ref):
    a = at_ref[:]
    ne = jnp.zeros((BN, ND), jnp.float32)
    for t in range(NTYPES):
        sel = (a == t).astype(jnp.float32)
        ne = ne + sel * tt_ref[t:t + 1, :]
    ne_ref[:] = ne
    p_ref[:] = _dot(ne, w2_ref[:])


def _stage0(at2, type_table, w2):
    return pl.pallas_call(
        _stage0_body,
        grid=(NB,),
        in_specs=[
            pl.BlockSpec((BN, 1), lambda i: (i, 0)),
            pl.BlockSpec((NTYPES, ND), lambda i: (0, 0)),
            pl.BlockSpec((ND, ED), lambda i: (0, 0)),
        ],
        out_specs=[
            pl.BlockSpec((BN, ND), lambda i: (i, 0)),
            pl.BlockSpec((BN, ED), lambda i: (i, 0)),
        ],
        out_shape=[
            jax.ShapeDtypeStruct((NLOC, ND), jnp.float32),
            jax.ShapeDtypeStruct((NLOC, ED), jnp.float32),
        ],
    )(at2, type_table, w2)


def _geom_body(t_ref, c_ref, we_ref, be_ref, e_ref, sw_ref):
    R3 = t_ref[:, ED:ED + CPAD].reshape(BN, NNEI, CPAD)
    C3 = c_ref[:].reshape(BN, CPAD, 1)
    d2 = lax.dot_general(R3, C3, (((2,), (1,)), ((0,), (0,))),
                         preferred_element_type=jnp.float32)
    dist = jnp.sqrt(jnp.maximum(d2, 0.0))
    uu = jnp.clip((dist - RCUT_SMTH) / (RCUT - RCUT_SMTH), 0.0, 1.0)
    sw = uu * uu * uu * (-6.0 * uu * uu + 15.0 * uu - 10.0) + 1.0
    we3 = we_ref[:].reshape(1, 1, ED)
    be3 = be_ref[:].reshape(1, 1, ED)
    e0 = _silu(dist * we3 + be3) * sw
    e_ref[:] = e0.reshape(BE, ED)
    sw_ref[:] = sw.reshape(BE, 1)


def _geom(t0g, cgeo, we0, be0):
    return pl.pallas_call(
        _geom_body,
        grid=(NB,),
        in_specs=[
            pl.BlockSpec((BE, ND), lambda i: (i, 0)),
            pl.BlockSpec((BN, CPAD), lambda i: (i, 0)),
            pl.BlockSpec((1, ED), lambda i: (0, 0)),
            pl.BlockSpec((1, ED), lambda i: (0, 0)),
        ],
        out_specs=[
            pl.BlockSpec((BE, ED), lambda i: (i, 0)),
            pl.BlockSpec((BE, 1), lambda i: (i, 0)),
        ],
        out_shape=[
            jax.ShapeDtypeStruct((E, ED), jnp.float32),
            jax.ShapeDtypeStruct((E, 1), jnp.float32),
        ],
    )(t0g, cgeo, we0, be0)


def _edge_body(last, g_direct, e_ref, nb_ref, n_ref, sw_ref, w1_ref, w2_ref,
               w3_ref, wn1_ref, wn2_ref, *out_refs):
    e = e_ref[:]
    sw = sw_ref[:]
    if g_direct:
        pre = _dot(e, w3_ref[:]) + nb_ref[:, :ED]
    else:
        pre = _dot(e, w3_ref[:]) + _dot(nb_ref[:], w2_ref[:])
    n = n_ref[:]
    a = _dot(n, w1_ref[:])
    sw3 = sw.reshape(BN, NNEI, 1)
    pre3 = pre.reshape(BN, NNEI, ED) + a[:, None, :]
    e3 = e.reshape(BN, NNEI, ED) + _silu(pre3) * sw3
    if last:
        (no_ref,) = out_refs
    else:
        eo_ref, no_ref = out_refs
        eo_ref[:] = e3.reshape(BE, ED)
    msg = jnp.sum(e3 * sw3, axis=1) * (1.0 / NNEI)
    h = _dot(n, wn1_ref[:]) + _dot(msg, wn2_ref[:])
    no_ref[:] = n + _silu(h)


def _edge(e, nbg, node, sw, w1, w2, w3, wn1, wn2, last, g_direct=False):
    out_specs = [pl.BlockSpec((BN, ND), lambda i: (i, 0))]
    out_shape = [jax.ShapeDtypeStruct((NLOC, ND), jnp.float32)]
    if not last:
        out_specs.insert(0, pl.BlockSpec((BE, ED), lambda i: (i, 0)))
        out_shape.insert(0, jax.ShapeDtypeStruct((E, ED), jnp.float32))
    return pl.pallas_call(
        functools.partial(_edge_body, last, g_direct),
        grid=(NB,),
        in_specs=[
            pl.BlockSpec((BE, ED), lambda i: (i, 0)),
            pl.BlockSpec((BE, ND), lambda i: (i, 0)),
            pl.BlockSpec((BN, ND), lambda i: (i, 0)),
            pl.BlockSpec((BE, 1), lambda i: (i, 0)),
            pl.BlockSpec((ND, ED), lambda i: (0, 0)),
            pl.BlockSpec((ND, ED), lambda i: (0, 0)),
            pl.BlockSpec((ED, ED), lambda i: (0, 0)),
            pl.BlockSpec((ND, ND), lambda i: (0, 0)),
            pl.BlockSpec((ED, ND), lambda i: (0, 0)),
        ],
        out_specs=out_specs,
        out_shape=out_shape,
    )(e, nbg, node, sw, w1, w2, w3, wn1, wn2)


def kernel(extended_coord, extended_atype, nlist, mapping, type_table,
           W_e0, b_e0, W_node, W_edge):
    coords = extended_coord[0].astype(jnp.float32)
    ones = jnp.ones((NLOC, 1), jnp.float32)
    zeros11 = jnp.zeros((NLOC, CPAD - 5), jnp.float32)
    cn2 = jnp.sum(coords * coords, axis=1, keepdims=True)
    csum = jnp.sum(coords, axis=1, keepdims=True)
    rgeo = jnp.concatenate([coords, cn2, ones, zeros11], axis=1)
    cgeo = jnp.concatenate(
        [-2.0 * coords + 2.0 * EPS, ones,
         cn2 - 2.0 * EPS * csum + 3.0 * EPS * EPS, zeros11], axis=1)
    at2 = extended_atype[0].astype(jnp.int32).reshape(NLOC, 1)
    idx3 = nlist[0].astype(jnp.int32).reshape(NW, RPW, RW)
    W1 = W_edge[:, :ND, :]
    W2 = W_edge[:, ND:2 * ND, :]
    W3 = W_edge[:, 2 * ND:, :]
    Wn1 = W_node[:, :ND, :]
    Wn2 = W_node[:, ND:, :]
    we0 = W_e0.reshape(1, ED)
    be0 = b_e0.reshape(1, ED)

    node0, p0 = _stage0(at2, type_table, W2[0])
    t0 = jnp.concatenate(
        [p0, rgeo, jnp.zeros((NLOC, ND - ED - CPAD), jnp.float32)], axis=1)
    t0g = _sc_gather(t0, idx3)
    e0, sw = _geom(t0g, cgeo, we0, be0)
    e1, node1 = _edge(e0, t0g, node0, sw, W1[0], W2[0], W3[0],
                      Wn1[0], Wn2[0], last=False, g_direct=True)
    nb1 = _sc_gather(node1, idx3)
    e2, node2 = _edge(e1, nb1, node1, sw, W1[1], W2[1], W3[1],
                      Wn1[1], Wn2[1], last=False)
    nb2 = _sc_gather(node2, idx3)
    (node3,) = _edge(e2, nb2, node2, sw, W1[2], W2[2], W3[2],
                     Wn1[2], Wn2[2], last=True)
    return node3[None]


# speedup vs baseline: 2.7357x; 1.0335x over previous
import functools

import jax
import jax.numpy as jnp
from jax import lax
from jax.experimental import pallas as pl
from jax.experimental.pallas import tpu as pltpu
from jax.experimental.pallas import tpu_sc as plsc

NTYPES = 8
NLOC = 10000
NNEI = 64
ND = 128
ED = 64
RCUT = 6.0
RCUT_SMTH = 5.0
EPS = 1e-6
CPAD = 16

E = NLOC * NNEI
BN = 80
BE = BN * NNEI
NB = NLOC // BN

CH = 5
NLOC_C = NLOC // CH
E_C = E // CH
NB_C = NLOC_C // BN

RW = 80
NC, NS = 2, 16
NW = NC * NS
RPW = E_C // (NW * RW)
EP = E_C


def _silu(x):
    return x / (1.0 + jnp.exp(-x))


def _dot(a, b):
    return jnp.dot(a, b, preferred_element_type=jnp.float32)


def _sc_gather(table, idx3):
    mesh = plsc.VectorSubcoreMesh(core_axis_name="c", subcore_axis_name="s")

    @functools.partial(
        pl.kernel,
        mesh=mesh,
        out_type=jax.ShapeDtypeStruct((EP, ND), jnp.float32),
        scratch_types=[
            pltpu.VMEM((RPW, RW), jnp.int32),
            pltpu.VMEM((2, RW, ND), jnp.float32),
            pltpu.SemaphoreType.DMA,
            pltpu.SemaphoreType.DMA,
        ],
    )
    def gk(table_hbm, idx_hbm, out_hbm, idx_v, row_v, sem0, sem1):
        wid = lax.axis_index("s") * NC + lax.axis_index("c")
        base = wid * RPW
        pltpu.sync_copy(idx_hbm.at[wid], idx_v)

        def gath(j, b, sem):
            pltpu.async_copy(table_hbm.at[idx_v.at[j]], row_v.at[b], sem)

        def gwait(b, sem):
            pltpu.make_async_copy(
                table_hbm.at[idx_v.at[0]], row_v.at[b], sem).wait()

        def scat(j, b):
            pltpu.sync_copy(
                row_v.at[b], out_hbm.at[pl.ds((base + j) * RW, RW)])

        gath(0, 0, sem0)

        def body(t, carry):
            j = 2 * t
            gath(j + 1, 1, sem1)
            gwait(0, sem0)
            scat(j, 0)
            gath(j + 2, 0, sem0)
            gwait(1, sem1)
            scat(j + 1, 1)
            return carry

        lax.fori_loop(0, RPW // 2 - 1, body, 0)
        gath(RPW - 1, 1, sem1)
        gwait(0, sem0)
        scat(RPW - 2, 0)
        gwait(1, sem1)
        scat(RPW - 1, 1)

    return gk(table, idx3)


def _stage0_body(at_ref, tt_ref, w2_ref, ne_ref, p_ref):
    a = at_ref[:]
    ne = jnp.zeros((BN, ND), jnp.float32)
    for t in range(NTYPES):
        sel = (a == t).astype(jnp.float32)
        ne = ne + sel * tt_ref[t:t + 1, :]
    ne_ref[:] = ne
    p_ref[:] = _dot(ne, w2_ref[:])


def _stage0(at2, type_table, w2):
    return pl.pallas_call(
        _stage0_body,
        grid=(NB,),
        in_specs=[
            pl.BlockSpec((BN, 1), lambda i: (i, 0)),
            pl.BlockSpec((NTYPES, ND), lambda i: (0, 0)),
            pl.BlockSpec((ND, ED), lambda i: (0, 0)),
        ],
        out_specs=[
            pl.BlockSpec((BN, ND), lambda i: (i, 0)),
            pl.BlockSpec((BN, ED), lambda i: (i, 0)),
        ],
        out_shape=[
            jax.ShapeDtypeStruct((NLOC, ND), jnp.float32),
            jax.ShapeDtypeStruct((NLOC, ED), jnp.float32),
        ],
    )(at2, type_table, w2)


def _geom_body(t_ref, c_ref, we_ref, be_ref, e_ref, sw_ref):
    R3 = t_ref[:, ED:ED + CPAD].reshape(BN, NNEI, CPAD)
    C3 = c_ref[:].reshape(BN, CPAD, 1)
    d2 = lax.dot_general(R3, C3, (((2,), (1,)), ((0,), (0,))),
                         preferred_element_type=jnp.float32)
    dist = jnp.sqrt(jnp.maximum(d2, 0.0))
    uu = jnp.clip((dist - RCUT_SMTH) / (RCUT - RCUT_SMTH), 0.0, 1.0)
    sw = uu * uu * uu * (-6.0 * uu * uu + 15.0 * uu - 10.0) + 1.0
    we3 = we_ref[:].reshape(1, 1, ED)
    be3 = be_ref[:].reshape(1, 1, ED)
    e0 = _silu(dist * we3 + be3) * sw
    e_ref[:] = e0.reshape(BE, ED)
    sw_ref[:] = sw.reshape(BE, 1)


def _geom(t0g, cgeo, we0, be0, k):
    off = k * NB_C
    return pl.pallas_call(
        _geom_body,
        grid=(NB_C,),
        in_specs=[
            pl.BlockSpec((BE, ND), lambda i: (i, 0)),
            pl.BlockSpec((BN, CPAD), lambda i: (off + i, 0)),
            pl.BlockSpec((1, ED), lambda i: (0, 0)),
            pl.BlockSpec((1, ED), lambda i: (0, 0)),
        ],
        out_specs=[
            pl.BlockSpec((BE, ED), lambda i: (i, 0)),
            pl.BlockSpec((BE, 1), lambda i: (i, 0)),
        ],
        out_shape=[
            jax.ShapeDtypeStruct((E_C, ED), jnp.float32),
            jax.ShapeDtypeStruct((E_C, 1), jnp.float32),
        ],
    )(t0g, cgeo, we0, be0)


def _edge_body(last, g_direct, e_ref, nb_ref, n_ref, sw_ref, w1_ref, w2_ref,
               w3_ref, wn1_ref, wn2_ref, *out_refs):
    e = e_ref[:]
    sw = sw_ref[:]
    if g_direct:
        pre = _dot(e, w3_ref[:]) + nb_ref[:, :ED]
    else:
        pre = _dot(e, w3_ref[:]) + _dot(nb_ref[:], w2_ref[:])
    n = n_ref[:]
    a = _dot(n, w1_ref[:])
    sw3 = sw.reshape(BN, NNEI, 1)
    pre3 = pre.reshape(BN, NNEI, ED) + a[:, None, :]
    e3 = e.reshape(BN, NNEI, ED) + _silu(pre3) * sw3
    if last:
        (no_ref,) = out_refs
    else:
        eo_ref, no_ref = out_refs
        eo_ref[:] = e3.reshape(BE, ED)
    msg = jnp.sum(e3 * sw3, axis=1) * (1.0 / NNEI)
    h = _dot(n, wn1_ref[:]) + _dot(msg, wn2_ref[:])
    no_ref[:] = n + _silu(h)


def _edge(e, nbg, node, sw, w1, w2, w3, wn1, wn2, k, last, g_direct=False):
    off = k * NB_C
    out_specs = [pl.BlockSpec((BN, ND), lambda i: (i, 0))]
    out_shape = [jax.ShapeDtypeStruct((NLOC_C, ND), jnp.float32)]
    if not last:
        out_specs.insert(0, pl.BlockSpec((BE, ED), lambda i: (i, 0)))
        out_shape.insert(0, jax.ShapeDtypeStruct((E_C, ED), jnp.float32))
    return pl.pallas_call(
        functools.partial(_edge_body, last, g_direct),
        grid=(NB_C,),
        in_specs=[
            pl.BlockSpec((BE, ED), lambda i: (i, 0)),
            pl.BlockSpec((BE, ND), lambda i: (i, 0)),
            pl.BlockSpec((BN, ND), lambda i: (off + i, 0)),
            pl.BlockSpec((BE, 1), lambda i: (i, 0)),
            pl.BlockSpec((ND, ED), lambda i: (0, 0)),
            pl.BlockSpec((ND, ED), lambda i: (0, 0)),
            pl.BlockSpec((ED, ED), lambda i: (0, 0)),
            pl.BlockSpec((ND, ND), lambda i: (0, 0)),
            pl.BlockSpec((ED, ND), lambda i: (0, 0)),
        ],
        out_specs=out_specs,
        out_shape=out_shape,
    )(e, nbg, node, sw, w1, w2, w3, wn1, wn2)


def kernel(extended_coord, extended_atype, nlist, mapping, type_table,
           W_e0, b_e0, W_node, W_edge):
    coords = extended_coord[0].astype(jnp.float32)
    ones = jnp.ones((NLOC, 1), jnp.float32)
    zeros11 = jnp.zeros((NLOC, CPAD - 5), jnp.float32)
    cn2 = jnp.sum(coords * coords, axis=1, keepdims=True)
    csum = jnp.sum(coords, axis=1, keepdims=True)
    rgeo = jnp.concatenate([coords, cn2, ones, zeros11], axis=1)
    cgeo = jnp.concatenate(
        [-2.0 * coords + 2.0 * EPS, ones,
         cn2 - 2.0 * EPS * csum + 3.0 * EPS * EPS, zeros11], axis=1)
    at2 = extended_atype[0].astype(jnp.int32).reshape(NLOC, 1)
    idx4 = nlist[0].astype(jnp.int32).reshape(CH, NW, RPW, RW)
    W1 = W_edge[:, :ND, :]
    W2 = W_edge[:, ND:2 * ND, :]
    W3 = W_edge[:, 2 * ND:, :]
    Wn1 = W_node[:, :ND, :]
    Wn2 = W_node[:, ND:, :]
    we0 = W_e0.reshape(1, ED)
    be0 = b_e0.reshape(1, ED)

    node0, p0 = _stage0(at2, type_table, W2[0])
    t0 = jnp.concatenate(
        [p0, rgeo, jnp.zeros((NLOC, ND - ED - CPAD), jnp.float32)], axis=1)

    e1, sw, n1 = [], [], []
    for k in range(CH):
        t0g_k = _sc_gather(t0, idx4[k])
        e0_k, sw_k = _geom(t0g_k, cgeo, we0, be0, k)
        e1_k, n1_k = _edge(e0_k, t0g_k, node0, sw_k, W1[0], W2[0], W3[0],
                           Wn1[0], Wn2[0], k, last=False, g_direct=True)
        e1.append(e1_k)
        sw.append(sw_k)
        n1.append(n1_k)
    node1 = jnp.concatenate(n1, axis=0)

    e2, n2 = [], []
    for k in range(CH):
        nb1_k = _sc_gather(node1, idx4[k])
        e2_k, n2_k = _edge(e1[k], nb1_k, node1, sw[k], W1[1], W2[1], W3[1],
                           Wn1[1], Wn2[1], k, last=False)
        e2.append(e2_k)
        n2.append(n2_k)
    node2 = jnp.concatenate(n2, axis=0)

    n3 = []
    for k in range(CH):
        nb2_k = _sc_gather(node2, idx4[k])
        (n3_k,) = _edge(e2[k], nb2_k, node2, sw[k], W1[2], W2[2], W3[2],
                        Wn1[2], Wn2[2], k, last=True)
        n3.append(n3_k)
    node3 = jnp.concatenate(n3, axis=0)
    return node3[None]


# speedup vs baseline: 2.9129x; 1.0648x over previous
import functools

import jax
import jax.numpy as jnp
from jax import lax
from jax.experimental import pallas as pl
from jax.experimental.pallas import tpu as pltpu
from jax.experimental.pallas import tpu_sc as plsc

NTYPES = 8
NLOC = 10000
NNEI = 64
ND = 128
ED = 64
RCUT = 6.0
RCUT_SMTH = 5.0
EPS = 1e-6
CPAD = 16

E = NLOC * NNEI
BN = 80
BE = BN * NNEI
NB = NLOC // BN

CH = 5
NLOC_C = NLOC // CH
E_C = E // CH
NB_C = NLOC_C // BN

RW = 80
NC, NS = 2, 16
NW = NC * NS
RPW = E_C // (NW * RW)
EP = E_C


def _silu(x):
    return x / (1.0 + jnp.exp(-x))


def _dot(a, b):
    return jnp.dot(a, b, preferred_element_type=jnp.float32)


def _sc_gather(table, idx3):
    mesh = plsc.VectorSubcoreMesh(core_axis_name="c", subcore_axis_name="s")

    @functools.partial(
        pl.kernel,
        mesh=mesh,
        out_type=jax.ShapeDtypeStruct((EP, ND), jnp.float32),
        scratch_types=[
            pltpu.VMEM((RPW, RW), jnp.int32),
            pltpu.VMEM((2, RW, ND), jnp.float32),
            pltpu.SemaphoreType.DMA,
            pltpu.SemaphoreType.DMA,
        ],
    )
    def gk(table_hbm, idx_hbm, out_hbm, idx_v, row_v, sem0, sem1):
        wid = lax.axis_index("s") * NC + lax.axis_index("c")
        base = wid * RPW
        pltpu.sync_copy(idx_hbm.at[wid], idx_v)

        def gath(j, b, sem):
            pltpu.async_copy(table_hbm.at[idx_v.at[j]], row_v.at[b], sem)

        def gwait(b, sem):
            pltpu.make_async_copy(
                table_hbm.at[idx_v.at[0]], row_v.at[b], sem).wait()

        def scat(j, b):
            pltpu.sync_copy(
                row_v.at[b], out_hbm.at[pl.ds((base + j) * RW, RW)])

        gath(0, 0, sem0)

        def body(t, carry):
            j = 2 * t
            gath(j + 1, 1, sem1)
            gwait(0, sem0)
            scat(j, 0)
            gath(j + 2, 0, sem0)
            gwait(1, sem1)
            scat(j + 1, 1)
            return carry

        lax.fori_loop(0, RPW // 2 - 1, body, 0)
        gath(RPW - 1, 1, sem1)
        gwait(0, sem0)
        scat(RPW - 2, 0)
        gwait(1, sem1)
        scat(RPW - 1, 1)

    return gk(table, idx3)


def _stage0_body(at_ref, tt_ref, w2_ref, ne_ref, p_ref):
    a = at_ref[:]
    ne = jnp.zeros((BN, ND), jnp.float32)
    for t in range(NTYPES):
        sel = (a == t).astype(jnp.float32)
        ne = ne + sel * tt_ref[t:t + 1, :]
    ne_ref[:] = ne
    p_ref[:] = _dot(ne, w2_ref[:])


def _stage0(at2, type_table, w2):
    return pl.pallas_call(
        _stage0_body,
        grid=(NB,),
        in_specs=[
            pl.BlockSpec((BN, 1), lambda i: (i, 0)),
            pl.BlockSpec((NTYPES, ND), lambda i: (0, 0)),
            pl.BlockSpec((ND, ED), lambda i: (0, 0)),
        ],
        out_specs=[
            pl.BlockSpec((BN, ND), lambda i: (i, 0)),
            pl.BlockSpec((BN, ED), lambda i: (i, 0)),
        ],
        out_shape=[
            jax.ShapeDtypeStruct((NLOC, ND), jnp.float32),
            jax.ShapeDtypeStruct((NLOC, ED), jnp.float32),
        ],
    )(at2, type_table, w2)


def _geom_body(t_ref, c_ref, we_ref, be_ref, e_ref, sw_ref):
    R3 = t_ref[:, ED:ED + CPAD].reshape(BN, NNEI, CPAD)
    C3 = c_ref[:].reshape(BN, CPAD, 1)
    d2 = lax.dot_general(R3, C3, (((2,), (1,)), ((0,), (0,))),
                         preferred_element_type=jnp.float32)
    dist = jnp.sqrt(jnp.maximum(d2, 0.0))
    uu = jnp.clip((dist - RCUT_SMTH) / (RCUT - RCUT_SMTH), 0.0, 1.0)
    sw = uu * uu * uu * (-6.0 * uu * uu + 15.0 * uu - 10.0) + 1.0
    we3 = we_ref[:].reshape(1, 1, ED)
    be3 = be_ref[:].reshape(1, 1, ED)
    e0 = _silu(dist * we3 + be3) * sw
    e_ref[:] = e0.reshape(BE, ED)
    sw_ref[:] = sw.reshape(BE, 1)


def _geom(t0g, cgeo, we0, be0, k):
    off = k * NB_C
    return pl.pallas_call(
        _geom_body,
        grid=(NB_C,),
        in_specs=[
            pl.BlockSpec((BE, ND), lambda i: (i, 0)),
            pl.BlockSpec((BN, CPAD), lambda i: (off + i, 0)),
            pl.BlockSpec((1, ED), lambda i: (0, 0)),
            pl.BlockSpec((1, ED), lambda i: (0, 0)),
        ],
        out_specs=[
            pl.BlockSpec((BE, ED), lambda i: (i, 0)),
            pl.BlockSpec((BE, 1), lambda i: (i, 0)),
        ],
        out_shape=[
            jax.ShapeDtypeStruct((E_C, ED), jnp.float32),
            jax.ShapeDtypeStruct((E_C, 1), jnp.float32),
        ],
    )(t0g, cgeo, we0, be0)


def _edge_body(last, g_direct, e_ref, nb_ref, n_ref, sw_ref, w1_ref, w2_ref,
               w3_ref, wn1_ref, wn2_ref, *out_refs):
    e = e_ref[:].astype(jnp.float32)
    sw = sw_ref[:]
    if g_direct:
        pre = _dot(e, w3_ref[:]) + nb_ref[:, :ED]
    else:
        pre = _dot(e, w3_ref[:]) + _dot(nb_ref[:], w2_ref[:])
    n = n_ref[:]
    a = _dot(n, w1_ref[:])
    sw3 = sw.reshape(BN, NNEI, 1)
    pre3 = pre.reshape(BN, NNEI, ED) + a[:, None, :]
    e3 = e.reshape(BN, NNEI, ED) + _silu(pre3) * sw3
    if last:
        (no_ref,) = out_refs
    else:
        eo_ref, no_ref = out_refs
        eo_ref[:] = e3.reshape(BE, ED).astype(jnp.bfloat16)
    msg = jnp.sum(e3 * sw3, axis=1) * (1.0 / NNEI)
    h = _dot(n, wn1_ref[:]) + _dot(msg, wn2_ref[:])
    no_ref[:] = n + _silu(h)


def _edge(e, nbg, node, sw, w1, w2, w3, wn1, wn2, k, last, g_direct=False):
    off = k * NB_C
    out_specs = [pl.BlockSpec((BN, ND), lambda i: (i, 0))]
    out_shape = [jax.ShapeDtypeStruct((NLOC_C, ND), jnp.float32)]
    if not last:
        out_specs.insert(0, pl.BlockSpec((BE, ED), lambda i: (i, 0)))
        out_shape.insert(0, jax.ShapeDtypeStruct((E_C, ED), jnp.bfloat16))
    return pl.pallas_call(
        functools.partial(_edge_body, last, g_direct),
        grid=(NB_C,),
        in_specs=[
            pl.BlockSpec((BE, ED), lambda i: (i, 0)),
            pl.BlockSpec((BE, ND), lambda i: (i, 0)),
            pl.BlockSpec((BN, ND), lambda i: (off + i, 0)),
            pl.BlockSpec((BE, 1), lambda i: (i, 0)),
            pl.BlockSpec((ND, ED), lambda i: (0, 0)),
            pl.BlockSpec((ND, ED), lambda i: (0, 0)),
            pl.BlockSpec((ED, ED), lambda i: (0, 0)),
            pl.BlockSpec((ND, ND), lambda i: (0, 0)),
            pl.BlockSpec((ED, ND), lambda i: (0, 0)),
        ],
        out_specs=out_specs,
        out_shape=out_shape,
    )(e, nbg, node, sw, w1, w2, w3, wn1, wn2)


def kernel(extended_coord, extended_atype, nlist, mapping, type_table,
           W_e0, b_e0, W_node, W_edge):
    coords = extended_coord[0].astype(jnp.float32)
    ones = jnp.ones((NLOC, 1), jnp.float32)
    zeros11 = jnp.zeros((NLOC, CPAD - 5), jnp.float32)
    cn2 = jnp.sum(coords * coords, axis=1, keepdims=True)
    csum = jnp.sum(coords, axis=1, keepdims=True)
    rgeo = jnp.concatenate([coords, cn2, ones, zeros11], axis=1)
    cgeo = jnp.concatenate(
        [-2.0 * coords + 2.0 * EPS, ones,
         cn2 - 2.0 * EPS * csum + 3.0 * EPS * EPS, zeros11], axis=1)
    at2 = extended_atype[0].astype(jnp.int32).reshape(NLOC, 1)
    idx4 = nlist[0].astype(jnp.int32).reshape(CH, NW, RPW, RW)
    W1 = W_edge[:, :ND, :]
    W2 = W_edge[:, ND:2 * ND, :]
    W3 = W_edge[:, 2 * ND:, :]
    Wn1 = W_node[:, :ND, :]
    Wn2 = W_node[:, ND:, :]
    we0 = W_e0.reshape(1, ED)
    be0 = b_e0.reshape(1, ED)

    node0, p0 = _stage0(at2, type_table, W2[0])
    t0 = jnp.concatenate(
        [p0, rgeo, jnp.zeros((NLOC, ND - ED - CPAD), jnp.float32)], axis=1)

    e1, sw, n1 = [], [], []
    for k in range(CH):
        t0g_k = _sc_gather(t0, idx4[k])
        e0_k, sw_k = _geom(t0g_k, cgeo, we0, be0, k)
        e1_k, n1_k = _edge(e0_k, t0g_k, node0, sw_k, W1[0], W2[0], W3[0],
                           Wn1[0], Wn2[0], k, last=False, g_direct=True)
        e1.append(e1_k)
        sw.append(sw_k)
        n1.append(n1_k)
    node1 = jnp.concatenate(n1, axis=0)

    e2, n2 = [], []
    for k in range(CH):
        nb1_k = _sc_gather(node1, idx4[k])
        e2_k, n2_k = _edge(e1[k], nb1_k, node1, sw[k], W1[1], W2[1], W3[1],
                           Wn1[1], Wn2[1], k, last=False)
        e2.append(e2_k)
        n2.append(n2_k)
    node2 = jnp.concatenate(n2, axis=0)

    n3 = []
    for k in range(CH):
        nb2_k = _sc_gather(node2, idx4[k])
        (n3_k,) = _edge(e2[k], nb2_k, node2, sw[k], W1[2], W2[2], W3[2],
                        Wn1[2], Wn2[2], k, last=True)
        n3.append(n3_k)
    node3 = jnp.concatenate(n3, axis=0)
    return node3[None]
